# baseline (speedup 1.0000x reference)
import functools

import jax
import jax.numpy as jnp
from jax import lax
from jax.experimental import pallas as pl
from jax.experimental.pallas import tpu as pltpu
from jax.experimental.pallas import tpu_sc as plsc

_INFO = plsc.get_sparse_core_info()
_NC, _NS = _INFO.num_cores, _INFO.num_subcores
_NW = _NC * _NS


@functools.lru_cache(maxsize=None)
def _make_gather_dequant(V, B, L, NB):
    assert B % (_NW * NB) == 0
    b_per_w = B // _NW
    n_chunks = b_per_w // NB
    CH = NB * L
    mesh = plsc.VectorSubcoreMesh(core_axis_name="c", subcore_axis_name="s")

    @functools.partial(
        pl.kernel,
        mesh=mesh,
        compiler_params=pltpu.CompilerParams(
            needs_layout_passes=False, use_tc_tiling_on_sc=False),
        out_type=jax.ShapeDtypeStruct((B, L, 64), jnp.float32),
        scratch_types=[
            pltpu.VMEM((n_chunks, CH), jnp.int32),
            pltpu.VMEM((CH, 8), jnp.int32),
            pltpu.VMEM((CH, 64), jnp.float32),
            pltpu.VMEM((16,), jnp.float32),
            pltpu.SemaphoreType.DMA,
            pltpu.SemaphoreType.DMA,
        ],
    )
    def k(table_hbm, idx_hbm, lut_hbm, out_hbm, idx_v, rows_v, out_v, lut_v,
          gsem, osem):
        wid = lax.axis_index("s") * _NC + lax.axis_index("c")
        pltpu.sync_copy(lut_hbm, lut_v)
        pltpu.sync_copy(idx_hbm.at[wid], idx_v)

        iota = lax.iota(jnp.int32, 16)
        half = iota >> 3
        c_word = iota & 7
        d8 = 8 * c_word
        douts = [d8 + kk for kk in range(8)]

        for c in range(n_chunks):
            pltpu.async_copy(table_hbm.at[idx_v.at[c]], rows_v, gsem).wait()

            def pair_body(g, carry):
                words = plsc.load_gather(rows_v, [half + 2 * g, c_word])
                r_idx = half + 2 * g
                for kk in range(8):
                    sh = 8 * (kk // 2) + 4 * (1 - kk % 2)
                    codes = (words >> sh) & 15
                    vals = plsc.load_gather(lut_v, [codes])
                    plsc.store_scatter(out_v, [r_idx, douts[kk]], vals)
                return carry

            lax.fori_loop(0, CH // 2, pair_body, 0)

            b_base = wid * b_per_w + c * NB
            for i in range(NB):
                pltpu.async_copy(
                    out_v.at[pl.ds(i * L, L), :], out_hbm.at[b_base + i],
                    osem).wait()

    return k


def kernel(x, nf4_lut, absmax, weight_quant_packed):
    B, L = x.shape
    V, Dh = weight_quant_packed.shape
    NB = 16
    table = lax.bitcast_convert_type(
        weight_quant_packed.reshape(V, Dh // 4, 4), jnp.int32)
    idx3 = x.reshape(_NW, (B // (_NW * NB)), NB * L)
    scaled_lut = (nf4_lut * absmax).astype(jnp.float32)
    return _make_gather_dequant(V, B, L, NB)(table, idx3, scaled_lut)


# speedup vs baseline: 1.4137x; 1.4137x over previous
import functools

import jax
import jax.numpy as jnp
from jax import lax
from jax.experimental import pallas as pl
from jax.experimental.pallas import tpu as pltpu
from jax.experimental.pallas import tpu_sc as plsc

_INFO = plsc.get_sparse_core_info()
_NC, _NS = _INFO.num_cores, _INFO.num_subcores
_NW = _NC * _NS


@functools.lru_cache(maxsize=None)
def _make_gather_dequant(V, N, CH):
    assert N % (_NW * CH) == 0 and CH % 2 == 0
    n_chunks = N // (_NW * CH)
    b_per_w = N // _NW
    mesh = plsc.VectorSubcoreMesh(core_axis_name="c", subcore_axis_name="s")

    @functools.partial(
        pl.kernel,
        mesh=mesh,
        compiler_params=pltpu.CompilerParams(
            needs_layout_passes=False, use_tc_tiling_on_sc=False),
        out_type=jax.ShapeDtypeStruct((N // 2, 128), jnp.float32),
        scratch_types=[
            pltpu.VMEM((n_chunks, CH), jnp.int32),
            pltpu.VMEM((CH, 8), jnp.int32),
            pltpu.VMEM((CH, 8), jnp.int32),
            pltpu.VMEM((CH // 2, 128), jnp.float32),
            pltpu.VMEM((CH // 2, 128), jnp.float32),
            pltpu.VMEM((16,), jnp.float32),
            pltpu.SemaphoreType.DMA,
            pltpu.SemaphoreType.DMA,
            pltpu.SemaphoreType.DMA,
            pltpu.SemaphoreType.DMA,
        ],
    )
    def k(table_hbm, idx_hbm, lut_hbm, out_hbm, idx_v, rows0, rows1, out0,
          out1, lut_v, gsem0, gsem1, osem0, osem1):
        wid = lax.axis_index("s") * _NC + lax.axis_index("c")
        pltpu.sync_copy(lut_hbm, lut_v)
        pltpu.sync_copy(idx_hbm.at[wid], idx_v)

        rows = (rows0, rows1)
        outs = (out0, out1)
        gsems = (gsem0, gsem1)
        osems = (osem0, osem1)

        iota = lax.iota(jnp.int32, 16)
        half = iota >> 3
        c_word = iota & 7
        douts = [8 * c_word + kk + 64 * half for kk in range(8)]

        gh = [None] * n_chunks
        oh = [None] * n_chunks
        gh[0] = pltpu.async_copy(table_hbm.at[idx_v.at[0]], rows[0], gsems[0])
        for c in range(n_chunks):
            b = c % 2
            gh[c].wait()
            if c + 1 < n_chunks:
                gh[c + 1] = pltpu.async_copy(
                    table_hbm.at[idx_v.at[c + 1]], rows[1 - b], gsems[1 - b])
            if c >= 2:
                oh[c - 2].wait()
            rows_b = rows[b]
            out_b = outs[b]

            @plsc.parallel_loop(0, CH // 2, 1, unroll=4)
            def _(g):
                words = plsc.load_gather(rows_b, [half + 2 * g, c_word])
                gfull = jnp.full((16,), 0, jnp.int32) + g
                for kk in range(8):
                    sh = 8 * (kk // 2) + 4 * (1 - kk % 2)
                    codes = (words >> sh) & 15
                    vals = plsc.load_gather(lut_v, [codes])
                    plsc.store_scatter(out_b, [gfull, douts[kk]], vals)

            p_base = pl.multiple_of((wid * b_per_w + c * CH) // 2, CH // 2)
            oh[c] = pltpu.async_copy(
                out_b, out_hbm.at[pl.ds(p_base, CH // 2), :], osems[b])
        oh[n_chunks - 2].wait()
        oh[n_chunks - 1].wait()

    return k


def kernel(x, nf4_lut, absmax, weight_quant_packed):
    B, L = x.shape
    V, Dh = weight_quant_packed.shape
    D = 2 * Dh
    N = B * L
    CH = 640
    table = lax.bitcast_convert_type(
        weight_quant_packed.reshape(V, Dh // 4, 4), jnp.int32)
    idx3 = x.reshape(_NW, N // (_NW * CH), CH)
    scaled_lut = (nf4_lut * absmax).astype(jnp.float32)
    out2 = _make_gather_dequant(V, N, CH)(table, idx3, scaled_lut)
    return out2.reshape(B, L, D)


# speedup vs baseline: 2.4893x; 1.7609x over previous
import functools

import jax
import jax.numpy as jnp
from jax import lax
from jax.experimental import pallas as pl
from jax.experimental.pallas import tpu as pltpu
from jax.experimental.pallas import tpu_sc as plsc

_INFO = plsc.get_sparse_core_info()
_NC, _NS = _INFO.num_cores, _INFO.num_subcores
_NW = _NC * _NS


@functools.lru_cache(maxsize=None)
def _make_table_convert(V):
    RW_MAIN = 3136
    RW_LAST = V - 31 * RW_MAIN
    CH_MAIN, CH_LAST = RW_MAIN // 4, RW_LAST // 4
    assert CH_MAIN % 4 == 0 and CH_LAST % 4 == 0 and RW_LAST > 0
    mesh = plsc.VectorSubcoreMesh(core_axis_name="c", subcore_axis_name="s")

    @functools.partial(
        pl.kernel,
        mesh=mesh,
        compiler_params=pltpu.CompilerParams(
            needs_layout_passes=False, use_tc_tiling_on_sc=False),
        out_type=jax.ShapeDtypeStruct((V // 2, 16), jnp.int32),
        scratch_types=[
            pltpu.VMEM((CH_MAIN, 32), jnp.uint8),
            pltpu.VMEM((CH_MAIN // 2, 16), jnp.int32),
        ],
    )
    def ka(packed_hbm, out_hbm, u8v, o32):
        wid = lax.axis_index("s") * _NC + lax.axis_index("c")

        def convert(base_row, ch):
            view = u8v.bitcast(jnp.int32)
            for c in range(4):
                pltpu.sync_copy(
                    packed_hbm.at[pl.ds(base_row + c * ch, ch)],
                    u8v.at[pl.ds(0, ch)])

                @plsc.parallel_loop(0, ch // 2, 1, unroll=4)
                def _(i):
                    o32[i] = view[i >> 3, pl.ds((i & 7) * 16, 16)]

                pltpu.sync_copy(
                    o32.at[pl.ds(0, ch // 2)],
                    out_hbm.at[pl.ds((base_row + c * ch) // 2, ch // 2)])

        @pl.when(wid < 31)
        def _():
            convert(wid * RW_MAIN, CH_MAIN)

        @pl.when(wid == 31)
        def _():
            convert(31 * RW_MAIN, CH_LAST)

    return ka


@functools.lru_cache(maxsize=None)
def _make_gather_dequant(V, B, L):
    assert B % (_NW * 128) == 0 and L % 2 == 0
    BT = B // 128
    bt_per_w = BT // _NW
    assert bt_per_w == 1
    mesh = plsc.VectorSubcoreMesh(core_axis_name="c", subcore_axis_name="s")

    @functools.partial(
        pl.kernel,
        mesh=mesh,
        compiler_params=pltpu.CompilerParams(
            needs_layout_passes=False, use_tc_tiling_on_sc=False),
        out_type=jax.ShapeDtypeStruct((L, 8, BT, 8, 128), jnp.float32),
        scratch_types=[
            pltpu.VMEM((128, L), jnp.int32),
            pltpu.VMEM((128,), jnp.int32),
            pltpu.VMEM((128,), jnp.int32),
            pltpu.VMEM((128, 8), jnp.int32),
            pltpu.VMEM((128, 8), jnp.int32),
            pltpu.VMEM((8, 8, 128), jnp.float32),
            pltpu.VMEM((8, 8, 128), jnp.float32),
            pltpu.VMEM((16,), jnp.float32),
            pltpu.SemaphoreType.DMA,
            pltpu.SemaphoreType.DMA,
            pltpu.SemaphoreType.DMA,
            pltpu.SemaphoreType.DMA,
        ],
    )
    def k(table_hbm, x_hbm, lut_hbm, out_hbm, idx_v, icolA, icolB, rowsA,
          rowsB, outA, outB, lut_v, gsemA, gsemB, osemA, osemB):
        wid = lax.axis_index("s") * _NC + lax.axis_index("c")
        pltpu.sync_copy(lut_hbm, lut_v)
        pltpu.sync_copy(x_hbm.at[pl.ds(wid * 128, 128)], idx_v)

        iota = lax.iota(jnp.int32, 16)
        half = iota >> 3
        c_word = iota & 7
        kvecs = [jnp.full((16,), kk, jnp.int32) for kk in range(8)]

        def build_icol(icol, l):
            lv = kvecs[0] + l
            for j in range(8):
                icol[pl.ds(j * 16, 16)] = plsc.load_gather(
                    idx_v, [iota + j * 16, lv])

        def dequant(rows_b, out_b):
            @plsc.parallel_loop(0, 64, 1, unroll=4)
            def _(jp):
                words = plsc.load_gather(rows_b, [half + 2 * jp, c_word])
                bi = half + 2 * jp
                for kk in range(8):
                    sh = 8 * (kk // 2) + 4 * (1 - kk % 2)
                    codes = (words >> sh) & 15
                    vals = plsc.load_gather(lut_v, [codes])
                    plsc.store_scatter(out_b, [c_word, kvecs[kk], bi], vals)

        build_icol(icolA, 0)
        pltpu.async_copy(table_hbm.at[icolA], rowsA, gsemA)
        build_icol(icolB, 1)
        pltpu.async_copy(table_hbm.at[icolB], rowsB, gsemB)

        def half_step(i, l, icol, rows_b, out_b, gsem, osem, last_i):
            pltpu.make_async_copy(table_hbm.at[icol], rows_b, gsem).wait()

            @pl.when(i > 0)
            def _():
                pltpu.make_async_copy(out_b, out_hbm.at[0, :, wid], osem).wait()

            dequant(rows_b, out_b)
            pltpu.async_copy(out_b, out_hbm.at[l, :, wid], osem)

            @pl.when(i < last_i)
            def _():
                build_icol(icol, l + 2)
                pltpu.async_copy(table_hbm.at[icol], rows_b, gsem)

        def body(i, carry):
            half_step(i, 2 * i, icolA, rowsA, outA, gsemA, osemA, L // 2 - 1)
            half_step(i, 2 * i + 1, icolB, rowsB, outB, gsemB, osemB,
                      L // 2 - 1)
            return carry

        lax.fori_loop(0, L // 2, body, 0)
        pltpu.make_async_copy(outA, out_hbm.at[0, :, wid], osemA).wait()
        pltpu.make_async_copy(outB, out_hbm.at[0, :, wid], osemB).wait()

    return k


def kernel(x, nf4_lut, absmax, weight_quant_packed):
    B, L = x.shape
    V, Dh = weight_quant_packed.shape
    table = _make_table_convert(V)(weight_quant_packed).reshape(V, 8)
    scaled_lut = (nf4_lut * absmax).astype(jnp.float32)
    out5 = _make_gather_dequant(V, B, L)(table, x, scaled_lut)
    return out5.transpose(2, 4, 0, 1, 3).reshape(B, L, 2 * Dh)


# speedup vs baseline: 2.7571x; 1.1076x over previous
import functools

import jax
import jax.numpy as jnp
from jax import lax
from jax.experimental import pallas as pl
from jax.experimental.pallas import tpu as pltpu
from jax.experimental.pallas import tpu_sc as plsc

_INFO = plsc.get_sparse_core_info()
_NC, _NS = _INFO.num_cores, _INFO.num_subcores
_NW = _NC * _NS


@functools.lru_cache(maxsize=None)
def _make_table_convert(V):
    RW_MAIN = 3136
    RW_LAST = V - 31 * RW_MAIN
    CH_MAIN, CH_LAST = RW_MAIN // 4, RW_LAST // 4
    assert CH_MAIN % 4 == 0 and CH_LAST % 4 == 0 and RW_LAST > 0
    mesh = plsc.VectorSubcoreMesh(core_axis_name="c", subcore_axis_name="s")

    @functools.partial(
        pl.kernel,
        mesh=mesh,
        compiler_params=pltpu.CompilerParams(
            needs_layout_passes=False, use_tc_tiling_on_sc=False),
        out_type=jax.ShapeDtypeStruct((V // 2, 16), jnp.int32),
        scratch_types=[
            pltpu.VMEM((CH_MAIN, 32), jnp.uint8),
            pltpu.VMEM((CH_MAIN // 2, 16), jnp.int32),
        ],
    )
    def ka(packed_hbm, out_hbm, u8v, o32):
        wid = lax.axis_index("s") * _NC + lax.axis_index("c")

        def convert(base_row, ch):
            view = u8v.bitcast(jnp.int32)
            for c in range(4):
                pltpu.sync_copy(
                    packed_hbm.at[pl.ds(base_row + c * ch, ch)],
                    u8v.at[pl.ds(0, ch)])

                @plsc.parallel_loop(0, ch // 2, 1, unroll=4)
                def _(i):
                    o32[i] = view[i >> 3, pl.ds((i & 7) * 16, 16)]

                pltpu.sync_copy(
                    o32.at[pl.ds(0, ch // 2)],
                    out_hbm.at[pl.ds((base_row + c * ch) // 2, ch // 2)])

        @pl.when(wid < 31)
        def _():
            convert(wid * RW_MAIN, CH_MAIN)

        @pl.when(wid == 31)
        def _():
            convert(31 * RW_MAIN, CH_LAST)

    return ka


@functools.lru_cache(maxsize=None)
def _make_gather_dequant(V, B, L):
    assert B % (_NW * 128) == 0 and L % 2 == 0
    BT = B // 128
    bt_per_w = BT // _NW
    assert bt_per_w == 1
    mesh = plsc.VectorSubcoreMesh(core_axis_name="c", subcore_axis_name="s")

    @functools.partial(
        pl.kernel,
        mesh=mesh,
        compiler_params=pltpu.CompilerParams(
            needs_layout_passes=False, use_tc_tiling_on_sc=False),
        out_type=jax.ShapeDtypeStruct((L, 8, BT, 8, 128), jnp.float32),
        scratch_types=[
            pltpu.VMEM((128, L), jnp.int32),
            pltpu.VMEM((128,), jnp.int32),
            pltpu.VMEM((128,), jnp.int32),
            pltpu.VMEM((128, 8), jnp.int32),
            pltpu.VMEM((128, 8), jnp.int32),
            pltpu.VMEM((8, 8, 128), jnp.float32),
            pltpu.VMEM((8, 8, 128), jnp.float32),
            pltpu.VMEM((16,), jnp.float32),
            pltpu.SemaphoreType.DMA,
            pltpu.SemaphoreType.DMA,
            pltpu.SemaphoreType.DMA,
            pltpu.SemaphoreType.DMA,
        ],
    )
    def k(table_hbm, x_hbm, lut_hbm, out_hbm, idx_v, icolA, icolB, rowsA,
          rowsB, outA, outB, lut_v, gsemA, gsemB, osemA, osemB):
        wid = lax.axis_index("s") * _NC + lax.axis_index("c")
        pltpu.sync_copy(lut_hbm, lut_v)
        pltpu.sync_copy(x_hbm.at[pl.ds(wid * 128, 128)], idx_v)

        iota = lax.iota(jnp.int32, 16)
        half = iota >> 3
        c_word = iota & 7
        kvecs = [jnp.full((16,), kk, jnp.int32) for kk in range(8)]

        def build_icol(icol, l):
            lv = kvecs[0] + l
            for j in range(8):
                icol[pl.ds(j * 16, 16)] = plsc.load_gather(
                    idx_v, [iota + j * 16, lv])

        def dequant(rows_b, out_b):
            @plsc.parallel_loop(0, 64, 1, unroll=8)
            def _(jp):
                words = plsc.load_gather(rows_b, [half + 2 * jp, c_word])
                bi = half + 2 * jp
                for kk in range(8):
                    sh = 8 * (kk // 2) + 4 * (1 - kk % 2)
                    codes = (words >> sh) & 15
                    vals = plsc.load_gather(lut_v, [codes])
                    plsc.store_scatter(out_b, [c_word, kvecs[kk], bi], vals)

        build_icol(icolA, 0)
        pltpu.async_copy(table_hbm.at[icolA], rowsA, gsemA)
        build_icol(icolB, 1)
        pltpu.async_copy(table_hbm.at[icolB], rowsB, gsemB)

        def half_step(i, l, icol, rows_b, out_b, gsem, osem, last_i):
            pltpu.make_async_copy(table_hbm.at[icol], rows_b, gsem).wait()

            @pl.when(i > 0)
            def _():
                pltpu.make_async_copy(out_b, out_hbm.at[0, :, wid], osem).wait()

            dequant(rows_b, out_b)
            pltpu.async_copy(out_b, out_hbm.at[l, :, wid], osem)

            @pl.when(i < last_i)
            def _():
                build_icol(icol, l + 2)
                pltpu.async_copy(table_hbm.at[icol], rows_b, gsem)

        def body(i, carry):
            half_step(i, 2 * i, icolA, rowsA, outA, gsemA, osemA, L // 2 - 1)
            half_step(i, 2 * i + 1, icolB, rowsB, outB, gsemB, osemB,
                      L // 2 - 1)
            return carry

        lax.fori_loop(0, L // 2, body, 0)
        pltpu.make_async_copy(outA, out_hbm.at[0, :, wid], osemA).wait()
        pltpu.make_async_copy(outB, out_hbm.at[0, :, wid], osemB).wait()

    return k


def kernel(x, nf4_lut, absmax, weight_quant_packed):
    B, L = x.shape
    V, Dh = weight_quant_packed.shape
    table = _make_table_convert(V)(weight_quant_packed).reshape(V, 8)
    scaled_lut = (nf4_lut * absmax).astype(jnp.float32)
    out5 = _make_gather_dequant(V, B, L)(table, x, scaled_lut)
    return out5.transpose(2, 4, 0, 1, 3).reshape(B, L, 2 * Dh)
